# baseline (speedup 1.0000x reference)
import functools

import jax
import jax.numpy as jnp
from jax import lax
from jax.experimental import pallas as pl
from jax.experimental.pallas import tpu as pltpu
from jax.experimental.pallas import tpu_sc as plsc

N_NODES = 10000
N_EDGES = 320000
D_FEAT = 128
D_EDGE = 16

NC = 2
NS = 16
NW = NC * NS
CHUNK = 64
N_CHUNKS = N_EDGES // CHUNK
N_CHUNKS_PAD = 5120
MAXC = N_CHUNKS_PAD // NW
N_ACC = 10240
ROWS_PER_TILE = N_ACC // NS

_mesh = plsc.VectorSubcoreMesh(core_axis_name="c", subcore_axis_name="s",
                               num_cores=NC, num_subcores=NS)


@functools.partial(
    pl.kernel,
    mesh=_mesh,
    out_type=jax.ShapeDtypeStruct((NC * N_ACC, D_FEAT), jnp.float32),
    scratch_types=[
        pltpu.VMEM((CHUNK,), jnp.int32),
        pltpu.VMEM((CHUNK,), jnp.int32),
        pltpu.VMEM((CHUNK,), jnp.int32),
        pltpu.VMEM((CHUNK,), jnp.int32),
        pltpu.VMEM((2 * CHUNK, D_FEAT), jnp.float32),
        pltpu.VMEM((2 * CHUNK, D_FEAT), jnp.float32),
        pltpu.VMEM_SHARED((N_ACC, D_FEAT), jnp.float32),
        pltpu.SemaphoreType.DMA,
        pltpu.SemaphoreType.DMA,
    ],
)
def _sc_aggregate(nodes_hbm, senders_hbm, receivers_hbm, eproj_hbm,
                  out_n, sidx_a, ridx_a, sidx_b, ridx_b, buf_a, buf_b, acc_n,
                  sem_a, sem_b):
    c = lax.axis_index("c")
    s = lax.axis_index("s")
    wid = s * NC + c
    r0 = s * ROWS_PER_TILE
    n_hop = ROWS_PER_TILE // CHUNK
    iota16 = lax.iota(jnp.int32, 16)
    zf16 = jnp.zeros((16,), jnp.float32)

    def _fill_ramp(base):
        for m in range(CHUNK // 16):
            sidx_a[pl.ds(m * 16, 16)] = iota16 + (base + m * 16)

    def _zero_rows(i, _):
        for m in range(D_FEAT // 16):
            buf_a[i, pl.ds(m * 16, 16)] = zf16
        return _
    lax.fori_loop(0, CHUNK, _zero_rows, None)

    for k in range(n_hop):
        _fill_ramp(r0 + k * CHUNK)
        pltpu.sync_copy(buf_a.at[pl.ds(0, CHUNK)], acc_n.at[sidx_a])
    plsc.subcore_barrier()

    lo = wid * MAXC

    def _eoff(j):
        return jnp.minimum(j * CHUNK, N_EDGES - CHUNK)

    def _fire_idx(j, sidx, ridx, sem):
        b = j * CHUNK
        pltpu.async_copy(senders_hbm.at[pl.ds(b, CHUNK)], sidx, sem)
        pltpu.async_copy(receivers_hbm.at[pl.ds(b, CHUNK)], ridx, sem)

    def _wait_idx(j, sidx, ridx, sem):
        b = j * CHUNK
        pltpu.make_async_copy(senders_hbm.at[pl.ds(b, CHUNK)], sidx, sem).wait()
        pltpu.make_async_copy(receivers_hbm.at[pl.ds(b, CHUNK)], ridx, sem).wait()

    def _fire_data(j, sidx, buf, sem):
        pltpu.async_copy(nodes_hbm.at[sidx], buf.at[pl.ds(0, CHUNK)], sem)
        pltpu.async_copy(eproj_hbm.at[pl.ds(_eoff(j), CHUNK)],
                         buf.at[pl.ds(CHUNK, CHUNK)], sem)

    def _wait_data(j, sidx, buf, sem):
        pltpu.make_async_copy(nodes_hbm.at[sidx], buf.at[pl.ds(0, CHUNK)],
                              sem).wait()
        pltpu.make_async_copy(eproj_hbm.at[pl.ds(_eoff(j), CHUNK)],
                              buf.at[pl.ds(CHUNK, CHUNK)], sem).wait()

    def _scatter(ridx, buf):
        pltpu.sync_copy(buf.at[pl.ds(0, CHUNK)], acc_n.at[ridx], add=True)
        pltpu.sync_copy(buf.at[pl.ds(CHUNK, CHUNK)], acc_n.at[ridx], add=True)

    _fire_idx(lo, sidx_a, ridx_a, sem_a)
    _fire_idx(lo + 1, sidx_b, ridx_b, sem_b)

    def step(t, _):
        j0 = lo + 2 * t
        _wait_idx(j0, sidx_a, ridx_a, sem_a)
        _fire_data(j0, sidx_a, buf_a, sem_a)
        _wait_idx(j0 + 1, sidx_b, ridx_b, sem_b)
        _fire_data(j0 + 1, sidx_b, buf_b, sem_b)
        _wait_data(j0, sidx_a, buf_a, sem_a)
        _scatter(ridx_a, buf_a)

        @pl.when(2 * t + 2 < MAXC)
        def _():
            _fire_idx(j0 + 2, sidx_a, ridx_a, sem_a)

        _wait_data(j0 + 1, sidx_b, buf_b, sem_b)
        _scatter(ridx_b, buf_b)

        @pl.when(2 * t + 3 < MAXC)
        def _():
            _fire_idx(j0 + 3, sidx_b, ridx_b, sem_b)

        return _

    lax.fori_loop(0, MAXC // 2, step, None)
    plsc.subcore_barrier()

    ro = c * N_ACC + r0
    for k in range(n_hop):
        _fill_ramp(r0 + k * CHUNK)
        pltpu.sync_copy(acc_n.at[sidx_a], buf_a.at[pl.ds(0, CHUNK)])
        pltpu.sync_copy(buf_a.at[pl.ds(0, CHUNK)], out_n.at[pl.ds(ro + k * CHUNK, CHUNK)])


_EP_R = 4000


def _eproj_body(edges_ref, We_ref, out_ref):
    out_ref[...] = jnp.dot(edges_ref[...], We_ref[...],
                           preferred_element_type=jnp.float32)


def _eproj(edges, We):
    return pl.pallas_call(
        _eproj_body,
        grid=(N_EDGES // _EP_R,),
        in_specs=[
            pl.BlockSpec((_EP_R, D_EDGE), lambda i: (i, 0)),
            pl.BlockSpec((D_EDGE, D_FEAT), lambda i: (0, 0)),
        ],
        out_specs=pl.BlockSpec((_EP_R, D_FEAT), lambda i: (i, 0)),
        out_shape=jax.ShapeDtypeStruct((N_EDGES, D_FEAT), jnp.float32),
    )(edges, We)


_R = 1000


def _mlp_body(pn_ref, nodes_ref, W1_ref, b1_ref, W2_ref, b2_ref, out_ref):
    h = pn_ref[0] + pn_ref[1] + nodes_ref[...]
    a = jnp.maximum(jnp.dot(h, W1_ref[...], preferred_element_type=jnp.float32)
                    + b1_ref[...], 0.0)
    out_ref[...] = (jnp.dot(a, W2_ref[...], preferred_element_type=jnp.float32)
                    + b2_ref[...])


def _mlp(pn, nodes, W1, b1, W2, b2):
    return pl.pallas_call(
        _mlp_body,
        grid=(N_NODES // _R,),
        in_specs=[
            pl.BlockSpec((NC, _R, D_FEAT), lambda i: (0, i, 0)),
            pl.BlockSpec((_R, D_FEAT), lambda i: (i, 0)),
            pl.BlockSpec((D_FEAT, D_FEAT), lambda i: (0, 0)),
            pl.BlockSpec((1, D_FEAT), lambda i: (0, 0)),
            pl.BlockSpec((D_FEAT, D_FEAT), lambda i: (0, 0)),
            pl.BlockSpec((1, D_FEAT), lambda i: (0, 0)),
        ],
        out_specs=pl.BlockSpec((_R, D_FEAT), lambda i: (i, 0)),
        out_shape=jax.ShapeDtypeStruct((N_NODES, D_FEAT), jnp.float32),
    )(pn, nodes, W1, b1, W2, b2)


def kernel(nodes, senders, receivers, edges, We, W1, b1, W2, b2):
    npad = (N_CHUNKS_PAD - N_CHUNKS) * CHUNK
    senders = jnp.concatenate(
        [senders.astype(jnp.int32), jnp.zeros((npad,), jnp.int32)])
    receivers = jnp.concatenate(
        [receivers.astype(jnp.int32), jnp.full((npad,), N_ACC - 1, jnp.int32)])
    eproj = _eproj(edges, We)
    pn = _sc_aggregate(nodes, senders, receivers, eproj)
    pn = pn.reshape(NC, N_ACC, D_FEAT)
    return _mlp(pn, nodes, W1, b1.reshape(1, D_FEAT), W2, b2.reshape(1, D_FEAT))
